# baseline (speedup 1.0000x reference)
import functools

import jax
import jax.numpy as jnp
from jax import lax
from jax.experimental import pallas as pl
from jax.experimental.pallas import tpu as pltpu
from jax.experimental.pallas import tpu_sc as plsc

N = 10000
E = 320000
D = 128
VOCAB = 1000
T = 4
N_SLT = 4
N_SRT = 9

NC, NS = 2, 16
NW = NC * NS
NPAD = 10240
NIDX = NPAD * T
HBINS = NPAD * N_SLT
EPT = E // NW
CE = 80
NCH = EPT // CE
GCH = 128
GPT = NIDX // NW
GNCH = GPT // GCH

_mesh = plsc.VectorSubcoreMesh(
    core_axis_name="c", subcore_axis_name="s", num_cores=NC, num_subcores=NS)


def _wid():
    return lax.axis_index("s") * NC + lax.axis_index("c")


def _drain(hbm_ref, buf, sem):
    pltpu.make_async_copy(hbm_ref.at[pl.ds(0, buf.shape[0])], buf, sem).wait()


def _k1_body(table_h, tidx_h, dst_h, typ_h, out0_h, hist_h,
             tidx_v, dst_v, typ_v, rows0, rows1, ob0, ob1, hacc_v,
             isem, g0, g1, w0, w1):
    wid = _wid()
    wsems = (w0, w1)

    pltpu.async_copy(tidx_h.at[pl.ds(wid * GPT, GPT)], tidx_v, isem)
    pltpu.async_copy(dst_h.at[pl.ds(wid * EPT, EPT)], dst_v, isem)
    pltpu.async_copy(typ_h.at[pl.ds(wid * EPT, EPT)], typ_v, isem)

    def zero(i, _):
        hacc_v[pl.ds(i * 16, 16)] = jnp.zeros((16,), jnp.float32)
        return 0
    lax.fori_loop(0, HBINS // 16, zero, 0)

    _drain(tidx_h, tidx_v, isem)
    _drain(dst_h, dst_v, isem)
    _drain(typ_h, typ_v, isem)

    rows = (rows0, rows1)
    obs = (ob0, ob1)
    gs = (g0, g1)

    def start_g(i, b):
        @pl.when(i < GNCH)
        def _():
            pltpu.async_copy(
                table_h.at[tidx_v.at[pl.ds(i * GCH, GCH)]], rows[b], gs[b])

    def pool_write(i, b):
        _drain(table_h, rows[b], gs[b])
        rb = rows[b]
        o = obs[b]

        @pl.when(i >= 2)
        def _():
            pltpu.make_async_copy(
                out0_h.at[pl.ds(0, GCH // 4)], o, wsems[b]).wait()

        def prow(r, _):
            def pcol(c, _):
                s = (rb[4 * r, pl.ds(c * 16, 16)]
                     + rb[4 * r + 1, pl.ds(c * 16, 16)]
                     + rb[4 * r + 2, pl.ds(c * 16, 16)]
                     + rb[4 * r + 3, pl.ds(c * 16, 16)])
                o[r, pl.ds(c * 16, 16)] = s
                return 0
            lax.fori_loop(0, D // 16, pcol, 0)
            return 0
        lax.fori_loop(0, GCH // 4, prow, 0)
        base = wid * (GPT // 4) + i * (GCH // 4)
        pltpu.async_copy(o, out0_h.at[pl.ds(base, GCH // 4)], wsems[b])

    start_g(0, 0)

    def gbody(j, _):
        i0 = 2 * j
        start_g(i0 + 1, 1)
        pool_write(i0, 0)
        start_g(i0 + 2, 0)
        pool_write(i0 + 1, 1)
        return 0
    lax.fori_loop(0, GNCH // 2, gbody, 0)
    pltpu.make_async_copy(out0_h.at[pl.ds(0, GCH // 4)], obs[0], wsems[0]).wait()
    pltpu.make_async_copy(out0_h.at[pl.ds(0, GCH // 4)], obs[1], wsems[1]).wait()

    ones = jnp.ones((16,), jnp.float32)

    def hstep(j, _):
        dk = dst_v[pl.ds(j * 16, 16)]
        tk = typ_v[pl.ds(j * 16, 16)]
        plsc.addupdate_scatter(hacc_v, [tk * NPAD + dk], ones)
        return 0
    lax.fori_loop(0, EPT // 16, hstep, 0)
    pltpu.sync_copy(hacc_v, hist_h.at[wid])


_k1 = functools.partial(
    pl.kernel, _k1_body,
    out_type=(jax.ShapeDtypeStruct((NPAD, D), jnp.float32),
              jax.ShapeDtypeStruct((NW, HBINS), jnp.float32)),
    mesh=_mesh,
    compiler_params=pltpu.CompilerParams(needs_layout_passes=False),
    scratch_types=[
        pltpu.VMEM((GPT,), jnp.int32),
        pltpu.VMEM((EPT,), jnp.int32),
        pltpu.VMEM((EPT,), jnp.int32),
        pltpu.VMEM((GCH, D), jnp.float32),
        pltpu.VMEM((GCH, D), jnp.float32),
        pltpu.VMEM((GCH // 4, D), jnp.float32),
        pltpu.VMEM((GCH // 4, D), jnp.float32),
        pltpu.VMEM((HBINS,), jnp.float32),
        pltpu.SemaphoreType.DMA,
        pltpu.SemaphoreType.DMA,
        pltpu.SemaphoreType.DMA,
        pltpu.SemaphoreType.DMA,
        pltpu.SemaphoreType.DMA,
    ])()


def _k3_body(x_h, src_h, dst_h, ap_h, sacc,
             s0, s1, d0, d1, rows0, rows1, zb_v,
             i0s, i1s, g0, g1):
    cid = lax.axis_index("c")
    sid = lax.axis_index("s")
    wid = sid * NC + cid
    ebase = wid * EPT

    srcs = (s0, s1)
    dsts = (d0, d1)
    rows = (rows0, rows1)
    isems = (i0s, i1s)
    gsems = (g0, g1)

    def start_idx(i, b):
        @pl.when(i < NCH)
        def _():
            base = ebase + i * CE
            pltpu.async_copy(src_h.at[pl.ds(base, CE)], srcs[b], isems[b])
            pltpu.async_copy(dst_h.at[pl.ds(base, CE)], dsts[b], isems[b])

    def wait_idx(i, b):
        @pl.when(i < NCH)
        def _():
            _drain(src_h, srcs[b], isems[b])
            _drain(dst_h, dsts[b], isems[b])

    def start_g(i, b):
        @pl.when(i < NCH)
        def _():
            pltpu.async_copy(x_h.at[srcs[b]], rows[b], gsems[b])

    def zvb(i, _):
        def zrow(j, _):
            zb_v[i, pl.ds(j * 16, 16)] = jnp.zeros((16,), jnp.float32)
            return 0
        lax.fori_loop(0, D // 16, zrow, 0)
        return 0
    lax.fori_loop(0, 64, zvb, 0)
    rbase = sid * (NPAD // NS)

    def zs(i, _):
        pltpu.sync_copy(zb_v, sacc.at[pl.ds(rbase + i * 64, 64)])
        return 0
    lax.fori_loop(0, NPAD // NS // 64, zs, 0)
    plsc.subcore_barrier()

    start_idx(0, 0)
    wait_idx(0, 0)
    start_g(0, 0)
    start_idx(1, 1)

    def half(i, b):
        wait_idx(i + 1, 1 - b)
        start_g(i + 1, 1 - b)
        _drain(x_h, rows[b], gsems[b])
        pltpu.sync_copy(rows[b], sacc.at[dsts[b]], add=True)
        start_idx(i + 2, b)

    def ebody(j, _):
        i0 = 2 * j
        half(i0, 0)
        half(i0 + 1, 1)
        return 0
    lax.fori_loop(0, NCH // 2, ebody, 0)
    half(NCH - 1, 0)
    plsc.subcore_barrier()

    pltpu.sync_copy(sacc.at[pl.ds(rbase, NPAD // NS)],
                    ap_h.at[cid, pl.ds(rbase, NPAD // NS)])


_k3 = functools.partial(
    pl.kernel, _k3_body,
    out_type=jax.ShapeDtypeStruct((NC, NPAD, D), jnp.float32),
    mesh=_mesh,
    scratch_types=[
        pltpu.VMEM_SHARED((NPAD, D), jnp.float32),
        pltpu.VMEM((CE,), jnp.int32),
        pltpu.VMEM((CE,), jnp.int32),
        pltpu.VMEM((CE,), jnp.int32),
        pltpu.VMEM((CE,), jnp.int32),
        pltpu.VMEM((CE, D), jnp.float32),
        pltpu.VMEM((CE, D), jnp.float32),
        pltpu.VMEM((64, D), jnp.float32),
        pltpu.SemaphoreType.DMA,
        pltpu.SemaphoreType.DMA,
        pltpu.SemaphoreType.DMA,
        pltpu.SemaphoreType.DMA,
    ])()


def _k6_body(p_h, q_h, src_h, dst_h, ef_h,
             s0, s1, d0, d1, pb0, pb1, qb0, qb1, ob0, ob1,
             i0s, i1s, gp0, gp1, gq0, gq1, w0, w1):
    wid = _wid()
    ebase = wid * EPT

    srcs = (s0, s1)
    dsts = (d0, d1)
    pbs = (pb0, pb1)
    qbs = (qb0, qb1)
    obs = (ob0, ob1)
    isems = (i0s, i1s)
    gpsems = (gp0, gp1)
    gqsems = (gq0, gq1)
    wsems = (w0, w1)

    def start_idx(i, b):
        @pl.when(i < NCH)
        def _():
            base = ebase + i * CE
            pltpu.async_copy(src_h.at[pl.ds(base, CE)], srcs[b], isems[b])
            pltpu.async_copy(dst_h.at[pl.ds(base, CE)], dsts[b], isems[b])

    def wait_idx(i, b):
        @pl.when(i < NCH)
        def _():
            _drain(src_h, srcs[b], isems[b])
            _drain(dst_h, dsts[b], isems[b])

    def start_g(i, b):
        @pl.when(i < NCH)
        def _():
            pltpu.async_copy(p_h.at[srcs[b]], pbs[b], gpsems[b])
            pltpu.async_copy(q_h.at[dsts[b]], qbs[b], gqsems[b])

    start_idx(0, 0)
    wait_idx(0, 0)
    start_g(0, 0)
    start_idx(1, 1)

    def half(i, b):
        wait_idx(i + 1, 1 - b)
        start_g(i + 1, 1 - b)
        _drain(p_h, pbs[b], gpsems[b])
        _drain(q_h, qbs[b], gqsems[b])
        pb = pbs[b]
        qb = qbs[b]
        o = obs[b]

        @pl.when(i >= 2)
        def _():
            pltpu.make_async_copy(
                ef_h.at[pl.ds(0, CE // 8)], o, wsems[b]).wait()

        def add(j, _):
            o[j // 8, pl.ds((j % 8) * 16, 16)] = pb[j] + qb[j]
            return 0
        lax.fori_loop(0, CE, add, 0)
        start_idx(i + 2, b)
        pltpu.async_copy(o, ef_h.at[pl.ds((ebase + i * CE) // 8, CE // 8)],
                         wsems[b])

    def ebody(j, _):
        i0 = 2 * j
        half(i0, 0)
        half(i0 + 1, 1)
        return 0
    lax.fori_loop(0, NCH // 2, ebody, 0)
    half(NCH - 1, 0)
    pltpu.make_async_copy(ef_h.at[pl.ds(0, CE // 8)], obs[1], wsems[1]).wait()
    pltpu.make_async_copy(ef_h.at[pl.ds(0, CE // 8)], obs[0], wsems[0]).wait()


_k6 = functools.partial(
    pl.kernel, _k6_body,
    out_type=jax.ShapeDtypeStruct((E * 16 // 128, 128), jnp.float32),
    mesh=_mesh,
    compiler_params=pltpu.CompilerParams(use_tc_tiling_on_sc=False),
    scratch_types=[
        pltpu.VMEM((CE,), jnp.int32),
        pltpu.VMEM((CE,), jnp.int32),
        pltpu.VMEM((CE,), jnp.int32),
        pltpu.VMEM((CE,), jnp.int32),
        pltpu.VMEM((CE, 16), jnp.float32),
        pltpu.VMEM((CE, 16), jnp.float32),
        pltpu.VMEM((CE, 16), jnp.float32),
        pltpu.VMEM((CE, 16), jnp.float32),
        pltpu.VMEM((CE // 8, 128), jnp.float32),
        pltpu.VMEM((CE // 8, 128), jnp.float32),
        pltpu.SemaphoreType.DMA,
        pltpu.SemaphoreType.DMA,
        pltpu.SemaphoreType.DMA,
        pltpu.SemaphoreType.DMA,
        pltpu.SemaphoreType.DMA,
        pltpu.SemaphoreType.DMA,
        pltpu.SemaphoreType.DMA,
        pltpu.SemaphoreType.DMA,
    ])()


EROWS = E * 16 // 128
RPT = EROWS // NW
RCH = CE // 8


def _k8_body(zp_h, out_h, zb0, zb1, pcls, g0, g1):
    wid = _wid()
    rbase = wid * RPT
    zbs = (zb0, zb1)
    gsems = (g0, g1)

    def start(i, b):
        @pl.when(i < NCH)
        def _():
            pltpu.async_copy(zp_h.at[pl.ds(rbase + i * RCH, RCH)],
                             zbs[b], gsems[b])

    start(0, 0)
    lane8 = jnp.arange(16, dtype=jnp.int32)
    rowv = lane8 // 8
    lanebase = (lane8 % 8) * 16

    def half(i, b):
        start(i + 1, 1 - b)
        _drain(zp_h, zbs[b], gsems[b])
        zb = zbs[b]
        for k in range(5):
            rv = rowv + 2 * k
            for c in range(N_SRT):
                g = plsc.load_gather(zb, [rv, lanebase + c])
                pcls[c, pl.ds(i * CE + 16 * k, 16)] = g

    def ebody(j, _):
        i0 = 2 * j
        half(i0, 0)
        half(i0 + 1, 1)
        return 0
    lax.fori_loop(0, NCH // 2, ebody, 0)
    half(NCH - 1, 0)

    for c in range(N_SRT):
        pltpu.sync_copy(pcls.at[c], out_h.at[c, pl.ds(wid * EPT, EPT)])


_k8 = functools.partial(
    pl.kernel, _k8_body,
    out_type=jax.ShapeDtypeStruct((N_SRT, E), jnp.float32),
    mesh=_mesh,
    compiler_params=pltpu.CompilerParams(
        needs_layout_passes=False, use_tc_tiling_on_sc=False),
    scratch_types=[
        pltpu.VMEM((RCH, 128), jnp.float32),
        pltpu.VMEM((RCH, 128), jnp.float32),
        pltpu.VMEM((N_SRT, EPT), jnp.float32),
        pltpu.SemaphoreType.DMA,
        pltpu.SemaphoreType.DMA,
    ])()


def _k2b_body(x_ref, hp_ref, re1_ref, re2_ref, wc1_ref, wc2_ref,
              b1_ref, b2_ref, ones_ref, add1_ref, add2_ref, invd_ref):
    cnt = jnp.sum(hp_ref[...], axis=0)
    dn = (((0,), (0,)), ((), ()))
    deg = lax.dot_general(cnt, ones_ref[...], dn,
                          preferred_element_type=jnp.float32)
    invd = 1.0 / jnp.maximum(deg, 1.0)
    mean_x = jnp.mean(x_ref[...], axis=0, keepdims=True)
    ctx1 = jnp.dot(mean_x, wc1_ref[...], preferred_element_type=jnp.float32)
    ctx2 = jnp.dot(mean_x, wc2_ref[...], preferred_element_type=jnp.float32)
    r1 = lax.dot_general(cnt, re1_ref[...], dn,
                         preferred_element_type=jnp.float32)
    r2 = lax.dot_general(cnt, re2_ref[...], dn,
                         preferred_element_type=jnp.float32)
    add1_ref[...] = r1 * invd + ctx1 + b1_ref[...][None, :]
    add2_ref[...] = r2 * invd + ctx2 + b2_ref[...][None, :]
    invd_ref[...] = invd


def _k2b(x, hp4, re1, re2, wc1, wc2, b1, b2):
    ones = jnp.ones((N_SLT, 1), jnp.float32)
    return pl.pallas_call(
        _k2b_body,
        out_shape=(jax.ShapeDtypeStruct((NPAD, D), jnp.float32),
                   jax.ShapeDtypeStruct((NPAD, D), jnp.float32),
                   jax.ShapeDtypeStruct((NPAD, 1), jnp.float32)),
    )(x, hp4, re1, re2, wc1, wc2, b1, b2, ones)


def _k4_body(xp_ref, ap_ref, invd_ref, addc_ref, ws_ref, wm_ref, out_ref):
    a = ap_ref[0] + ap_ref[1]
    h = jnp.dot(xp_ref[...], ws_ref[...], preferred_element_type=jnp.float32)
    m = jnp.dot(a, wm_ref[...], preferred_element_type=jnp.float32)
    out_ref[...] = jnp.maximum(h + m * invd_ref[...] + addc_ref[...], 0.0)


def _k4(xprev, ap, invd, addc, w_self, w_msg):
    blk = 256
    return pl.pallas_call(
        _k4_body,
        grid=(NPAD // blk,),
        in_specs=[
            pl.BlockSpec((blk, D), lambda i: (i, 0)),
            pl.BlockSpec((NC, blk, D), lambda i: (0, i, 0)),
            pl.BlockSpec((blk, 1), lambda i: (i, 0)),
            pl.BlockSpec((blk, D), lambda i: (i, 0)),
            pl.BlockSpec((D, D), lambda i: (0, 0)),
            pl.BlockSpec((D, D), lambda i: (0, 0)),
        ],
        out_specs=pl.BlockSpec((blk, D), lambda i: (i, 0)),
        out_shape=jax.ShapeDtypeStruct((NPAD, D), jnp.float32),
    )(xprev, ap, invd, addc, w_self, w_msg)


def _k4b_body(xp_ref, ap_ref, invd_ref, addc_ref, ws_ref, wm_ref,
              wg_ref, bg_ref, out_ref, pq_ref):
    a = ap_ref[0] + ap_ref[1]
    h = jnp.dot(xp_ref[...], ws_ref[...], preferred_element_type=jnp.float32)
    m = jnp.dot(a, wm_ref[...], preferred_element_type=jnp.float32)
    out = jnp.maximum(h + m * invd_ref[...] + addc_ref[...], 0.0)
    out_ref[...] = out
    pq = jnp.dot(out, wg_ref[...], preferred_element_type=jnp.float32)
    pq_ref[...] = pq + bg_ref[...][None, :]


def _k4b(xprev, ap, invd, addc, w_self, w_msg, wg_pad, bg_pad):
    blk = 256
    return pl.pallas_call(
        _k4b_body,
        grid=(NPAD // blk,),
        in_specs=[
            pl.BlockSpec((blk, D), lambda i: (i, 0)),
            pl.BlockSpec((NC, blk, D), lambda i: (0, i, 0)),
            pl.BlockSpec((blk, 1), lambda i: (i, 0)),
            pl.BlockSpec((blk, D), lambda i: (i, 0)),
            pl.BlockSpec((D, D), lambda i: (0, 0)),
            pl.BlockSpec((D, D), lambda i: (0, 0)),
            pl.BlockSpec((D, 32), lambda i: (0, 0)),
            pl.BlockSpec((32,), lambda i: (0,)),
        ],
        out_specs=(pl.BlockSpec((blk, D), lambda i: (i, 0)),
                   pl.BlockSpec((blk, 32), lambda i: (i, 0))),
        out_shape=(jax.ShapeDtypeStruct((NPAD, D), jnp.float32),
                   jax.ShapeDtypeStruct((NPAD, 32), jnp.float32)),
    )(xprev, ap, invd, addc, w_self, w_msg, wg_pad, bg_pad)


def _k5_body(x_ref, wz_ref, bz_ref, npt_ref):
    x = x_ref[...]
    dn = (((0,), (1,)), ((), ()))
    logits = lax.dot_general(wz_ref[...], x, dn,
                             preferred_element_type=jnp.float32)
    logits = logits + bz_ref[...]
    m = jnp.max(logits, axis=0, keepdims=True)
    z = logits - m
    ls = jnp.log(jnp.sum(jnp.exp(z), axis=0, keepdims=True))
    npt_ref[...] = z - ls


def _k5(out2, w_z, b_z):
    blk = 512
    grid = (N + blk - 1) // blk
    return pl.pallas_call(
        _k5_body,
        grid=(grid,),
        in_specs=[
            pl.BlockSpec((blk, D), lambda i: (i, 0)),
            pl.BlockSpec((D, VOCAB), lambda i: (0, 0)),
            pl.BlockSpec((VOCAB, 1), lambda i: (0, 0)),
        ],
        out_specs=pl.BlockSpec((VOCAB, blk), lambda i: (0, i)),
        out_shape=jax.ShapeDtypeStruct((VOCAB, N), jnp.float32),
    )(out2, w_z, b_z.reshape(VOCAB, 1))


def _k7_body(ef_ref, bmean_ref, bsum_ref, out_ref):
    z = ef_ref[...]
    lane = lax.broadcasted_iota(jnp.int32, z.shape, 1)
    valid = (lane % 16) < N_SRT
    zv = jnp.where(valid, z, 0.0)
    mean = jnp.dot(zv, bmean_ref[...], preferred_element_type=jnp.float32)
    zc = z - mean
    e = jnp.where(valid, jnp.exp(zc), 0.0)
    s = jnp.dot(e, bsum_ref[...], preferred_element_type=jnp.float32)
    out_ref[...] = zc - jnp.log(s)


def _k7(efp, bmean, bsum):
    blk = 4000
    return pl.pallas_call(
        _k7_body,
        grid=(EROWS // blk,),
        in_specs=[
            pl.BlockSpec((blk, 128), lambda i: (i, 0)),
            pl.BlockSpec((128, 128), lambda i: (0, 0)),
            pl.BlockSpec((128, 128), lambda i: (0, 0)),
        ],
        out_specs=pl.BlockSpec((blk, 128), lambda i: (i, 0)),
        out_shape=jax.ShapeDtypeStruct((EROWS, 128), jnp.float32),
    )(efp, bmean, bsum)


def kernel(x, tgt_x, tgt_edge_index, tgt_edge_type, embed_table,
           W_self1, W_msg1, W_ctx1, rel_emb1, b1,
           W_self2, W_msg2, W_ctx2, rel_emb2, b2,
           W_z, b_z, W_g, b_g):
    src = tgt_edge_index[0].astype(jnp.int32)
    dst = tgt_edge_index[1].astype(jnp.int32)
    typ = tgt_edge_type.astype(jnp.int32)
    tidx = jnp.concatenate(
        [tgt_x.astype(jnp.int32).reshape(-1),
         jnp.zeros((NIDX - N * T,), jnp.int32)])

    out0, hist = _k1(embed_table, tidx, dst, typ)
    hp4 = hist.reshape(NW, N_SLT, NPAD)
    add1, add2, invd = _k2b(x, hp4, rel_emb1, rel_emb2, W_ctx1, W_ctx2, b1, b2)

    wg_pad = jnp.zeros((D, 32), jnp.float32)
    wg_pad = wg_pad.at[:, :N_SRT].set(W_g[:D])
    wg_pad = wg_pad.at[:, 16:16 + N_SRT].set(W_g[D:])
    bg_pad = jnp.zeros((32,), jnp.float32).at[16:16 + N_SRT].set(b_g)

    ap1 = _k3(out0, src, dst)
    out1 = _k4(out0, ap1, invd, add1, W_self1, W_msg1)
    ap2 = _k3(out1, src, dst)
    out2, pq = _k4b(out1, ap2, invd, add2, W_self2, W_msg2, wg_pad, bg_pad)

    efp = _k6(pq[:, :16], pq[:, 16:], src, dst)
    npt = _k5(out2, W_z, b_z)
    node_pred = npt.T
    grp = jnp.arange(128, dtype=jnp.int32) // 16
    bgrp = (grp[:, None] == grp[None, :]).astype(jnp.float32)
    zp = _k7(efp, bgrp / float(N_SRT), bgrp)
    edge_pred = _k8(zp).T
    return (node_pred, edge_pred)


# speedup vs baseline: 1.0087x; 1.0087x over previous
import functools

import jax
import jax.numpy as jnp
from jax import lax
from jax.experimental import pallas as pl
from jax.experimental.pallas import tpu as pltpu
from jax.experimental.pallas import tpu_sc as plsc

N = 10000
E = 320000
D = 128
VOCAB = 1000
T = 4
N_SLT = 4
N_SRT = 9

NC, NS = 2, 16
NW = NC * NS
NPAD = 10240
NIDX = NPAD * T
HBINS = NPAD * N_SLT
EPT = E // NW
CE = 80
NCH = EPT // CE
GCH = 128
GPT = NIDX // NW
GNCH = GPT // GCH

_mesh = plsc.VectorSubcoreMesh(
    core_axis_name="c", subcore_axis_name="s", num_cores=NC, num_subcores=NS)


def _wid():
    return lax.axis_index("s") * NC + lax.axis_index("c")


def _drain(hbm_ref, buf, sem):
    pltpu.make_async_copy(hbm_ref.at[pl.ds(0, buf.shape[0])], buf, sem).wait()


def _k1_body(table_h, tidx_h, dst_h, typ_h, out0_h, hist_h,
             tidx_v, dst_v, typ_v, rows0, rows1, ob0, ob1, hacc_v,
             isem, g0, g1, w0, w1):
    wid = _wid()
    wsems = (w0, w1)

    pltpu.async_copy(tidx_h.at[pl.ds(wid * GPT, GPT)], tidx_v, isem)
    pltpu.async_copy(dst_h.at[pl.ds(wid * EPT, EPT)], dst_v, isem)
    pltpu.async_copy(typ_h.at[pl.ds(wid * EPT, EPT)], typ_v, isem)

    def zero(i, _):
        hacc_v[pl.ds(i * 16, 16)] = jnp.zeros((16,), jnp.float32)
        return 0
    lax.fori_loop(0, HBINS // 16, zero, 0)

    _drain(tidx_h, tidx_v, isem)
    _drain(dst_h, dst_v, isem)
    _drain(typ_h, typ_v, isem)

    rows = (rows0, rows1)
    obs = (ob0, ob1)
    gs = (g0, g1)

    def start_g(i, b):
        @pl.when(i < GNCH)
        def _():
            pltpu.async_copy(
                table_h.at[tidx_v.at[pl.ds(i * GCH, GCH)]], rows[b], gs[b])

    def pool_write(i, b):
        _drain(table_h, rows[b], gs[b])
        rb = rows[b]
        o = obs[b]

        @pl.when(i >= 2)
        def _():
            pltpu.make_async_copy(
                out0_h.at[pl.ds(0, GCH // 4)], o, wsems[b]).wait()

        def prow(r, _):
            def pcol(c, _):
                s = (rb[4 * r, pl.ds(c * 16, 16)]
                     + rb[4 * r + 1, pl.ds(c * 16, 16)]
                     + rb[4 * r + 2, pl.ds(c * 16, 16)]
                     + rb[4 * r + 3, pl.ds(c * 16, 16)])
                o[r, pl.ds(c * 16, 16)] = s
                return 0
            lax.fori_loop(0, D // 16, pcol, 0)
            return 0
        lax.fori_loop(0, GCH // 4, prow, 0)
        base = wid * (GPT // 4) + i * (GCH // 4)
        pltpu.async_copy(o, out0_h.at[pl.ds(base, GCH // 4)], wsems[b])

    start_g(0, 0)

    ones = jnp.ones((16,), jnp.float32)
    HPG = EPT // 16 // (GNCH // 2)

    def hstep(j, _):
        dk = dst_v[pl.ds(j * 16, 16)]
        tk = typ_v[pl.ds(j * 16, 16)]
        plsc.addupdate_scatter(hacc_v, [tk * NPAD + dk], ones)
        return 0

    def gbody(j, _):
        i0 = 2 * j
        start_g(i0 + 1, 1)
        lax.fori_loop(j * HPG, (j + 1) * HPG, hstep, 0)
        pool_write(i0, 0)
        start_g(i0 + 2, 0)
        pool_write(i0 + 1, 1)
        return 0
    lax.fori_loop(0, GNCH // 2, gbody, 0)
    pltpu.make_async_copy(out0_h.at[pl.ds(0, GCH // 4)], obs[0], wsems[0]).wait()
    pltpu.make_async_copy(out0_h.at[pl.ds(0, GCH // 4)], obs[1], wsems[1]).wait()
    pltpu.sync_copy(hacc_v, hist_h.at[wid])


_k1 = functools.partial(
    pl.kernel, _k1_body,
    out_type=(jax.ShapeDtypeStruct((NPAD, D), jnp.float32),
              jax.ShapeDtypeStruct((NW, HBINS), jnp.float32)),
    mesh=_mesh,
    compiler_params=pltpu.CompilerParams(needs_layout_passes=False),
    scratch_types=[
        pltpu.VMEM((GPT,), jnp.int32),
        pltpu.VMEM((EPT,), jnp.int32),
        pltpu.VMEM((EPT,), jnp.int32),
        pltpu.VMEM((GCH, D), jnp.float32),
        pltpu.VMEM((GCH, D), jnp.float32),
        pltpu.VMEM((GCH // 4, D), jnp.float32),
        pltpu.VMEM((GCH // 4, D), jnp.float32),
        pltpu.VMEM((HBINS,), jnp.float32),
        pltpu.SemaphoreType.DMA,
        pltpu.SemaphoreType.DMA,
        pltpu.SemaphoreType.DMA,
        pltpu.SemaphoreType.DMA,
        pltpu.SemaphoreType.DMA,
    ])()


def _k3_body(x_h, src_h, dst_h, ap_h, sacc,
             s0, s1, d0, d1, rows0, rows1, zb_v,
             i0s, i1s, g0, g1):
    cid = lax.axis_index("c")
    sid = lax.axis_index("s")
    wid = sid * NC + cid
    ebase = wid * EPT

    srcs = (s0, s1)
    dsts = (d0, d1)
    rows = (rows0, rows1)
    isems = (i0s, i1s)
    gsems = (g0, g1)

    def start_idx(i, b):
        @pl.when(i < NCH)
        def _():
            base = ebase + i * CE
            pltpu.async_copy(src_h.at[pl.ds(base, CE)], srcs[b], isems[b])
            pltpu.async_copy(dst_h.at[pl.ds(base, CE)], dsts[b], isems[b])

    def wait_idx(i, b):
        @pl.when(i < NCH)
        def _():
            _drain(src_h, srcs[b], isems[b])
            _drain(dst_h, dsts[b], isems[b])

    def start_g(i, b):
        @pl.when(i < NCH)
        def _():
            pltpu.async_copy(x_h.at[srcs[b]], rows[b], gsems[b])

    def zvb(i, _):
        def zrow(j, _):
            zb_v[i, pl.ds(j * 16, 16)] = jnp.zeros((16,), jnp.float32)
            return 0
        lax.fori_loop(0, D // 16, zrow, 0)
        return 0
    lax.fori_loop(0, 64, zvb, 0)
    rbase = sid * (NPAD // NS)

    def zs(i, _):
        pltpu.sync_copy(zb_v, sacc.at[pl.ds(rbase + i * 64, 64)])
        return 0
    lax.fori_loop(0, NPAD // NS // 64, zs, 0)
    plsc.subcore_barrier()

    start_idx(0, 0)
    wait_idx(0, 0)
    start_g(0, 0)
    start_idx(1, 1)

    def half(i, b):
        wait_idx(i + 1, 1 - b)
        start_g(i + 1, 1 - b)
        _drain(x_h, rows[b], gsems[b])
        pltpu.sync_copy(rows[b], sacc.at[dsts[b]], add=True)
        start_idx(i + 2, b)

    def ebody(j, _):
        i0 = 2 * j
        half(i0, 0)
        half(i0 + 1, 1)
        return 0
    lax.fori_loop(0, NCH // 2, ebody, 0)
    half(NCH - 1, 0)
    plsc.subcore_barrier()

    pltpu.sync_copy(sacc.at[pl.ds(rbase, NPAD // NS)],
                    ap_h.at[cid, pl.ds(rbase, NPAD // NS)])


_k3 = functools.partial(
    pl.kernel, _k3_body,
    out_type=jax.ShapeDtypeStruct((NC, NPAD, D), jnp.float32),
    mesh=_mesh,
    scratch_types=[
        pltpu.VMEM_SHARED((NPAD, D), jnp.float32),
        pltpu.VMEM((CE,), jnp.int32),
        pltpu.VMEM((CE,), jnp.int32),
        pltpu.VMEM((CE,), jnp.int32),
        pltpu.VMEM((CE,), jnp.int32),
        pltpu.VMEM((CE, D), jnp.float32),
        pltpu.VMEM((CE, D), jnp.float32),
        pltpu.VMEM((64, D), jnp.float32),
        pltpu.SemaphoreType.DMA,
        pltpu.SemaphoreType.DMA,
        pltpu.SemaphoreType.DMA,
        pltpu.SemaphoreType.DMA,
    ])()


def _k6_body(p_h, q_h, src_h, dst_h, ef_h,
             s0, s1, d0, d1, pb0, pb1, qb0, qb1, ob0, ob1,
             i0s, i1s, gp0, gp1, gq0, gq1, w0, w1):
    wid = _wid()
    ebase = wid * EPT

    srcs = (s0, s1)
    dsts = (d0, d1)
    pbs = (pb0, pb1)
    qbs = (qb0, qb1)
    obs = (ob0, ob1)
    isems = (i0s, i1s)
    gpsems = (gp0, gp1)
    gqsems = (gq0, gq1)
    wsems = (w0, w1)

    def start_idx(i, b):
        @pl.when(i < NCH)
        def _():
            base = ebase + i * CE
            pltpu.async_copy(src_h.at[pl.ds(base, CE)], srcs[b], isems[b])
            pltpu.async_copy(dst_h.at[pl.ds(base, CE)], dsts[b], isems[b])

    def wait_idx(i, b):
        @pl.when(i < NCH)
        def _():
            _drain(src_h, srcs[b], isems[b])
            _drain(dst_h, dsts[b], isems[b])

    def start_g(i, b):
        @pl.when(i < NCH)
        def _():
            pltpu.async_copy(p_h.at[srcs[b]], pbs[b], gpsems[b])
            pltpu.async_copy(q_h.at[dsts[b]], qbs[b], gqsems[b])

    start_idx(0, 0)
    wait_idx(0, 0)
    start_g(0, 0)
    start_idx(1, 1)

    def half(i, b):
        wait_idx(i + 1, 1 - b)
        start_g(i + 1, 1 - b)
        _drain(p_h, pbs[b], gpsems[b])
        _drain(q_h, qbs[b], gqsems[b])
        pb = pbs[b]
        qb = qbs[b]
        o = obs[b]

        @pl.when(i >= 2)
        def _():
            pltpu.make_async_copy(
                ef_h.at[pl.ds(0, CE // 8)], o, wsems[b]).wait()

        def add(j, _):
            o[j // 8, pl.ds((j % 8) * 16, 16)] = pb[j] + qb[j]
            return 0
        lax.fori_loop(0, CE, add, 0)
        start_idx(i + 2, b)
        pltpu.async_copy(o, ef_h.at[pl.ds((ebase + i * CE) // 8, CE // 8)],
                         wsems[b])

    def ebody(j, _):
        i0 = 2 * j
        half(i0, 0)
        half(i0 + 1, 1)
        return 0
    lax.fori_loop(0, NCH // 2, ebody, 0)
    half(NCH - 1, 0)
    pltpu.make_async_copy(ef_h.at[pl.ds(0, CE // 8)], obs[1], wsems[1]).wait()
    pltpu.make_async_copy(ef_h.at[pl.ds(0, CE // 8)], obs[0], wsems[0]).wait()


_k6 = functools.partial(
    pl.kernel, _k6_body,
    out_type=jax.ShapeDtypeStruct((E * 16 // 128, 128), jnp.float32),
    mesh=_mesh,
    compiler_params=pltpu.CompilerParams(use_tc_tiling_on_sc=False),
    scratch_types=[
        pltpu.VMEM((CE,), jnp.int32),
        pltpu.VMEM((CE,), jnp.int32),
        pltpu.VMEM((CE,), jnp.int32),
        pltpu.VMEM((CE,), jnp.int32),
        pltpu.VMEM((CE, 16), jnp.float32),
        pltpu.VMEM((CE, 16), jnp.float32),
        pltpu.VMEM((CE, 16), jnp.float32),
        pltpu.VMEM((CE, 16), jnp.float32),
        pltpu.VMEM((CE // 8, 128), jnp.float32),
        pltpu.VMEM((CE // 8, 128), jnp.float32),
        pltpu.SemaphoreType.DMA,
        pltpu.SemaphoreType.DMA,
        pltpu.SemaphoreType.DMA,
        pltpu.SemaphoreType.DMA,
        pltpu.SemaphoreType.DMA,
        pltpu.SemaphoreType.DMA,
        pltpu.SemaphoreType.DMA,
        pltpu.SemaphoreType.DMA,
    ])()


EROWS = E * 16 // 128
RPT = EROWS // NW
RCH = CE // 8


def _k8_body(zp_h, out_h, zb0, zb1, pcls, g0, g1):
    wid = _wid()
    rbase = wid * RPT
    zbs = (zb0, zb1)
    gsems = (g0, g1)

    def start(i, b):
        @pl.when(i < NCH)
        def _():
            pltpu.async_copy(zp_h.at[pl.ds(rbase + i * RCH, RCH)],
                             zbs[b], gsems[b])

    start(0, 0)
    lane8 = jnp.arange(16, dtype=jnp.int32)
    rowv = lane8 // 8
    lanebase = (lane8 % 8) * 16

    def half(i, b):
        start(i + 1, 1 - b)
        _drain(zp_h, zbs[b], gsems[b])
        zb = zbs[b]
        for k in range(5):
            rv = rowv + 2 * k
            for c in range(N_SRT):
                g = plsc.load_gather(zb, [rv, lanebase + c])
                pcls[c, pl.ds(i * CE + 16 * k, 16)] = g

    def ebody(j, _):
        i0 = 2 * j
        half(i0, 0)
        half(i0 + 1, 1)
        return 0
    lax.fori_loop(0, NCH // 2, ebody, 0)
    half(NCH - 1, 0)

    for c in range(N_SRT):
        pltpu.sync_copy(pcls.at[c], out_h.at[c, pl.ds(wid * EPT, EPT)])


_k8 = functools.partial(
    pl.kernel, _k8_body,
    out_type=jax.ShapeDtypeStruct((N_SRT, E), jnp.float32),
    mesh=_mesh,
    compiler_params=pltpu.CompilerParams(
        needs_layout_passes=False, use_tc_tiling_on_sc=False),
    scratch_types=[
        pltpu.VMEM((RCH, 128), jnp.float32),
        pltpu.VMEM((RCH, 128), jnp.float32),
        pltpu.VMEM((N_SRT, EPT), jnp.float32),
        pltpu.SemaphoreType.DMA,
        pltpu.SemaphoreType.DMA,
    ])()


def _k2b_body(x_ref, hp_ref, re1_ref, re2_ref, wc1_ref, wc2_ref,
              b1_ref, b2_ref, ones_ref, add1_ref, add2_ref, invd_ref):
    cnt = jnp.sum(hp_ref[...], axis=0)
    dn = (((0,), (0,)), ((), ()))
    deg = lax.dot_general(cnt, ones_ref[...], dn,
                          preferred_element_type=jnp.float32)
    invd = 1.0 / jnp.maximum(deg, 1.0)
    mean_x = jnp.mean(x_ref[...], axis=0, keepdims=True)
    ctx1 = jnp.dot(mean_x, wc1_ref[...], preferred_element_type=jnp.float32)
    ctx2 = jnp.dot(mean_x, wc2_ref[...], preferred_element_type=jnp.float32)
    r1 = lax.dot_general(cnt, re1_ref[...], dn,
                         preferred_element_type=jnp.float32)
    r2 = lax.dot_general(cnt, re2_ref[...], dn,
                         preferred_element_type=jnp.float32)
    add1_ref[...] = r1 * invd + ctx1 + b1_ref[...][None, :]
    add2_ref[...] = r2 * invd + ctx2 + b2_ref[...][None, :]
    invd_ref[...] = invd


def _k2b(x, hp4, re1, re2, wc1, wc2, b1, b2):
    ones = jnp.ones((N_SLT, 1), jnp.float32)
    return pl.pallas_call(
        _k2b_body,
        out_shape=(jax.ShapeDtypeStruct((NPAD, D), jnp.float32),
                   jax.ShapeDtypeStruct((NPAD, D), jnp.float32),
                   jax.ShapeDtypeStruct((NPAD, 1), jnp.float32)),
    )(x, hp4, re1, re2, wc1, wc2, b1, b2, ones)


def _k4_body(xp_ref, ap_ref, invd_ref, addc_ref, ws_ref, wm_ref, out_ref):
    a = ap_ref[0] + ap_ref[1]
    h = jnp.dot(xp_ref[...], ws_ref[...], preferred_element_type=jnp.float32)
    m = jnp.dot(a, wm_ref[...], preferred_element_type=jnp.float32)
    out_ref[...] = jnp.maximum(h + m * invd_ref[...] + addc_ref[...], 0.0)


def _k4(xprev, ap, invd, addc, w_self, w_msg):
    blk = 256
    return pl.pallas_call(
        _k4_body,
        grid=(NPAD // blk,),
        in_specs=[
            pl.BlockSpec((blk, D), lambda i: (i, 0)),
            pl.BlockSpec((NC, blk, D), lambda i: (0, i, 0)),
            pl.BlockSpec((blk, 1), lambda i: (i, 0)),
            pl.BlockSpec((blk, D), lambda i: (i, 0)),
            pl.BlockSpec((D, D), lambda i: (0, 0)),
            pl.BlockSpec((D, D), lambda i: (0, 0)),
        ],
        out_specs=pl.BlockSpec((blk, D), lambda i: (i, 0)),
        out_shape=jax.ShapeDtypeStruct((NPAD, D), jnp.float32),
    )(xprev, ap, invd, addc, w_self, w_msg)


def _k4b_body(xp_ref, ap_ref, invd_ref, addc_ref, ws_ref, wm_ref,
              wg_ref, bg_ref, out_ref, pq_ref):
    a = ap_ref[0] + ap_ref[1]
    h = jnp.dot(xp_ref[...], ws_ref[...], preferred_element_type=jnp.float32)
    m = jnp.dot(a, wm_ref[...], preferred_element_type=jnp.float32)
    out = jnp.maximum(h + m * invd_ref[...] + addc_ref[...], 0.0)
    out_ref[...] = out
    pq = jnp.dot(out, wg_ref[...], preferred_element_type=jnp.float32)
    pq_ref[...] = pq + bg_ref[...][None, :]


def _k4b(xprev, ap, invd, addc, w_self, w_msg, wg_pad, bg_pad):
    blk = 256
    return pl.pallas_call(
        _k4b_body,
        grid=(NPAD // blk,),
        in_specs=[
            pl.BlockSpec((blk, D), lambda i: (i, 0)),
            pl.BlockSpec((NC, blk, D), lambda i: (0, i, 0)),
            pl.BlockSpec((blk, 1), lambda i: (i, 0)),
            pl.BlockSpec((blk, D), lambda i: (i, 0)),
            pl.BlockSpec((D, D), lambda i: (0, 0)),
            pl.BlockSpec((D, D), lambda i: (0, 0)),
            pl.BlockSpec((D, 32), lambda i: (0, 0)),
            pl.BlockSpec((32,), lambda i: (0,)),
        ],
        out_specs=(pl.BlockSpec((blk, D), lambda i: (i, 0)),
                   pl.BlockSpec((blk, 32), lambda i: (i, 0))),
        out_shape=(jax.ShapeDtypeStruct((NPAD, D), jnp.float32),
                   jax.ShapeDtypeStruct((NPAD, 32), jnp.float32)),
    )(xprev, ap, invd, addc, w_self, w_msg, wg_pad, bg_pad)


def _k5_body(x_ref, wz_ref, bz_ref, npt_ref):
    x = x_ref[...]
    dn = (((0,), (1,)), ((), ()))
    logits = lax.dot_general(wz_ref[...], x, dn,
                             preferred_element_type=jnp.float32)
    logits = logits + bz_ref[...]
    m = jnp.max(logits, axis=0, keepdims=True)
    z = logits - m
    ls = jnp.log(jnp.sum(jnp.exp(z), axis=0, keepdims=True))
    npt_ref[...] = z - ls


def _k5(out2, w_z, b_z):
    blk = 512
    grid = (N + blk - 1) // blk
    return pl.pallas_call(
        _k5_body,
        grid=(grid,),
        in_specs=[
            pl.BlockSpec((blk, D), lambda i: (i, 0)),
            pl.BlockSpec((D, VOCAB), lambda i: (0, 0)),
            pl.BlockSpec((VOCAB, 1), lambda i: (0, 0)),
        ],
        out_specs=pl.BlockSpec((VOCAB, blk), lambda i: (0, i)),
        out_shape=jax.ShapeDtypeStruct((VOCAB, N), jnp.float32),
    )(out2, w_z, b_z.reshape(VOCAB, 1))


def _k7_body(ef_ref, bmean_ref, bsum_ref, out_ref):
    z = ef_ref[...]
    lane = lax.broadcasted_iota(jnp.int32, z.shape, 1)
    valid = (lane % 16) < N_SRT
    zv = jnp.where(valid, z, 0.0)
    mean = jnp.dot(zv, bmean_ref[...], preferred_element_type=jnp.float32)
    zc = z - mean
    e = jnp.where(valid, jnp.exp(zc), 0.0)
    s = jnp.dot(e, bsum_ref[...], preferred_element_type=jnp.float32)
    out_ref[...] = zc - jnp.log(s)


def _k7(efp, bmean, bsum):
    blk = 4000
    return pl.pallas_call(
        _k7_body,
        grid=(EROWS // blk,),
        in_specs=[
            pl.BlockSpec((blk, 128), lambda i: (i, 0)),
            pl.BlockSpec((128, 128), lambda i: (0, 0)),
            pl.BlockSpec((128, 128), lambda i: (0, 0)),
        ],
        out_specs=pl.BlockSpec((blk, 128), lambda i: (i, 0)),
        out_shape=jax.ShapeDtypeStruct((EROWS, 128), jnp.float32),
    )(efp, bmean, bsum)


def kernel(x, tgt_x, tgt_edge_index, tgt_edge_type, embed_table,
           W_self1, W_msg1, W_ctx1, rel_emb1, b1,
           W_self2, W_msg2, W_ctx2, rel_emb2, b2,
           W_z, b_z, W_g, b_g):
    src = tgt_edge_index[0].astype(jnp.int32)
    dst = tgt_edge_index[1].astype(jnp.int32)
    typ = tgt_edge_type.astype(jnp.int32)
    tidx = jnp.concatenate(
        [tgt_x.astype(jnp.int32).reshape(-1),
         jnp.zeros((NIDX - N * T,), jnp.int32)])

    out0, hist = _k1(embed_table, tidx, dst, typ)
    hp4 = hist.reshape(NW, N_SLT, NPAD)
    add1, add2, invd = _k2b(x, hp4, rel_emb1, rel_emb2, W_ctx1, W_ctx2, b1, b2)

    wg_pad = jnp.zeros((D, 32), jnp.float32)
    wg_pad = wg_pad.at[:, :N_SRT].set(W_g[:D])
    wg_pad = wg_pad.at[:, 16:16 + N_SRT].set(W_g[D:])
    bg_pad = jnp.zeros((32,), jnp.float32).at[16:16 + N_SRT].set(b_g)

    ap1 = _k3(out0, src, dst)
    out1 = _k4(out0, ap1, invd, add1, W_self1, W_msg1)
    ap2 = _k3(out1, src, dst)
    out2, pq = _k4b(out1, ap2, invd, add2, W_self2, W_msg2, wg_pad, bg_pad)

    efp = _k6(pq[:, :16], pq[:, 16:], src, dst)
    npt = _k5(out2, W_z, b_z)
    node_pred = npt.T
    grp = jnp.arange(128, dtype=jnp.int32) // 16
    bgrp = (grp[:, None] == grp[None, :]).astype(jnp.float32)
    zp = _k7(efp, bgrp / float(N_SRT), bgrp)
    edge_pred = _k8(zp).T
    return (node_pred, edge_pred)
